# speedup vs baseline: 2.5306x; 2.5306x over previous
import functools
import math

import jax
import jax.numpy as jnp
from jax import lax
from jax.experimental import pallas as pl
from jax.experimental.pallas import tpu as pltpu
from jax.experimental.pallas import tpu_sc as plsc

_VOCAB = 100000
_D = 128
_BATCH = 4096
_SEQ = 50
_B = _BATCH * _SEQ
_SCALE = math.sqrt(_D)

_NC = 2
_NS = 16
_NW = _NC * _NS
_BPW = _B // _NW
_CHUNK = 320
_NCHUNK = _BPW // _CHUNK


@functools.partial(
    pl.kernel,
    mesh=plsc.VectorSubcoreMesh(core_axis_name="c", subcore_axis_name="s"),
    out_type=jax.ShapeDtypeStruct((_B, _D), jnp.float32),
    scratch_types=[
        pltpu.VMEM((_CHUNK,), jnp.int32),
        pltpu.VMEM((_CHUNK, _D), jnp.float32),
        pltpu.SemaphoreType.DMA,
    ],
)
def _emb_lookup(tok_hbm, table_hbm, out_hbm, idx_v, rows_v, sem):
    wid = lax.axis_index("s") * _NC + lax.axis_index("c")
    base = wid * _BPW

    def chunk_body(g, carry):
        row0 = base + g * _CHUNK
        pltpu.sync_copy(tok_hbm.at[pl.ds(row0, _CHUNK)], idx_v)
        pltpu.async_copy(table_hbm.at[idx_v], rows_v, sem).wait()

        def scale_row(i, c):
            for j in range(_D // 16):
                sl = pl.ds(j * 16, 16)
                rows_v[i, sl] = rows_v[i, sl] * _SCALE
            return c

        lax.fori_loop(0, _CHUNK, scale_row, 0)
        pltpu.sync_copy(rows_v, out_hbm.at[pl.ds(row0, _CHUNK)])
        return carry

    lax.fori_loop(0, _NCHUNK, chunk_body, 0)


def kernel(text_tokens, table):
    flat_tok = text_tokens.reshape(_B).astype(jnp.int32)
    out = _emb_lookup(flat_tok, table)
    return out.reshape(_BATCH, _SEQ, _D)


# speedup vs baseline: 2.9017x; 1.1466x over previous
import functools
import math

import jax
import jax.numpy as jnp
from jax import lax
from jax.experimental import pallas as pl
from jax.experimental.pallas import tpu as pltpu
from jax.experimental.pallas import tpu_sc as plsc

_VOCAB = 100000
_D = 128
_BATCH = 4096
_SEQ = 50
_B = _BATCH * _SEQ
_SCALE = math.sqrt(_D)

_NC = 2
_NS = 16
_NW = _NC * _NS
_BPW = _B // _NW
_CHUNK = 320
_NCHUNK = _BPW // _CHUNK


@functools.partial(
    pl.kernel,
    mesh=plsc.VectorSubcoreMesh(core_axis_name="c", subcore_axis_name="s"),
    out_type=jax.ShapeDtypeStruct((_B, _D), jnp.float32),
    scratch_types=[
        pltpu.VMEM((_CHUNK,), jnp.int32),
        pltpu.VMEM((_CHUNK,), jnp.int32),
        pltpu.VMEM((_CHUNK, _D), jnp.float32),
        pltpu.VMEM((_CHUNK, _D), jnp.float32),
        pltpu.SemaphoreType.DMA,
        pltpu.SemaphoreType.DMA,
        pltpu.SemaphoreType.DMA,
        pltpu.SemaphoreType.DMA,
    ],
)
def _emb_lookup(tok_hbm, table_hbm, out_hbm, idx0, idx1, rows0, rows1,
                gsem0, gsem1, osem0, osem1):
    idx = (idx0, idx1)
    rows = (rows0, rows1)
    gsem = (gsem0, gsem1)
    osem = (osem0, osem1)

    wid = lax.axis_index("s") * _NC + lax.axis_index("c")
    base = wid * _BPW

    def start_gather(g, b):
        pltpu.sync_copy(tok_hbm.at[pl.ds(base + g * _CHUNK, _CHUNK)], idx[b])
        pltpu.async_copy(table_hbm.at[idx[b]], rows[b], gsem[b])

    def wait_gather(b):
        pltpu.make_async_copy(
            table_hbm.at[pl.ds(0, _CHUNK)], rows[b], gsem[b]).wait()

    def wait_out(b):
        pltpu.make_async_copy(
            rows[b], out_hbm.at[pl.ds(0, _CHUNK)], osem[b]).wait()

    def scale_buf(b):
        def scale_pair(i, c):
            for u in range(2):
                for j in range(_D // 16):
                    sl = pl.ds(j * 16, 16)
                    rows[b][i * 2 + u, sl] = rows[b][i * 2 + u, sl] * _SCALE
            return c
        lax.fori_loop(0, _CHUNK // 2, scale_pair, 0)

    start_gather(0, 0)

    def pair_body(p, carry):
        g0 = p * 2
        for b in range(2):
            g = g0 + b
            nb = 1 - b
            @pl.when(g >= 1)
            def _():
                wait_out(nb)

            @pl.when(g + 1 < _NCHUNK)
            def _():
                start_gather(g + 1, nb)

            wait_gather(b)
            scale_buf(b)
            pltpu.async_copy(
                rows[b], out_hbm.at[pl.ds(base + g * _CHUNK, _CHUNK)], osem[b])
        return carry

    lax.fori_loop(0, _NCHUNK // 2, pair_body, 0)
    wait_out(1)


def kernel(text_tokens, table):
    flat_tok = text_tokens.reshape(_B).astype(jnp.int32)
    out = _emb_lookup(flat_tok, table)
    return out.reshape(_BATCH, _SEQ, _D)
